# baseline (speedup 1.0000x reference)
import functools

import jax
import jax.numpy as jnp
from jax import lax
from jax.experimental import pallas as pl
from jax.experimental.pallas import tpu as pltpu
from jax.experimental.pallas import tpu_sc as plsc

_NUM_SPEAKERS = 100000
_DIM = 64
_BATCH = 16384
_DPAD = 128


@functools.cache
def _make_gather(V, D, B):
    info = plsc.get_sparse_core_info()
    NC, NS = info.num_cores, info.num_subcores
    NW = NC * NS
    assert B % NW == 0
    b_per_w = B // NW
    mesh = plsc.VectorSubcoreMesh(core_axis_name="c", subcore_axis_name="s")

    @functools.partial(
        pl.kernel,
        mesh=mesh,
        out_type=jax.ShapeDtypeStruct((B, _DPAD), jnp.float32),
        scratch_types=[
            pltpu.VMEM((b_per_w,), jnp.int32),
            pltpu.VMEM((4, b_per_w // 4, D), jnp.float32),
            pltpu.SemaphoreType.DMA,
            pltpu.SemaphoreType.DMA,
        ],
        compiler_params=pltpu.CompilerParams(
            skip_device_barrier=True, use_tc_tiling_on_sc=False
        ),
    )
    def gather_kernel(table_hbm, idx_hbm, out_hbm, idx_v, rows_v, gsem, wsem):
        wid = lax.axis_index("s") * NC + lax.axis_index("c")
        base = wid * b_per_w
        pltpu.sync_copy(idx_hbm.at[pl.ds(base, b_per_w)], idx_v)
        q = b_per_w // 4
        gathers = [
            pltpu.async_copy(
                table_hbm.at[idx_v.at[pl.ds(k * q, q)]], rows_v.at[k], gsem
            )
            for k in range(4)
        ]
        writes = []
        for k in range(4):
            gathers[k].wait()
            writes.append(
                pltpu.async_copy(
                    rows_v.at[k],
                    out_hbm.at[pl.ds(base + k * q, q), pl.ds(0, D)],
                    wsem,
                )
            )
        for w in writes:
            w.wait()

    return gather_kernel


@jax.jit
def kernel(spk_ids, table):
    gather = _make_gather(_NUM_SPEAKERS, _DIM, _BATCH)
    table_pad = jnp.pad(table, ((0, 0), (0, _DPAD - _DIM)))
    table_half = table_pad.reshape(2 * _NUM_SPEAKERS, _DIM)
    out_pad = gather(table_half, spk_ids.astype(jnp.int32) << 1)
    return out_pad[:, :_DIM]


# speedup vs baseline: 1.0053x; 1.0053x over previous
import functools

import jax
import jax.numpy as jnp
from jax import lax
from jax.experimental import pallas as pl
from jax.experimental.pallas import tpu as pltpu
from jax.experimental.pallas import tpu_sc as plsc

_NUM_SPEAKERS = 100000
_DIM = 64
_BATCH = 16384
_DPAD = 128


@functools.cache
def _make_gather(V, D, B):
    info = plsc.get_sparse_core_info()
    NC, NS = info.num_cores, info.num_subcores
    NW = NC * NS
    assert B % NW == 0
    b_per_w = B // NW
    mesh = plsc.VectorSubcoreMesh(core_axis_name="c", subcore_axis_name="s")

    @functools.partial(
        pl.kernel,
        mesh=mesh,
        out_type=jax.ShapeDtypeStruct((B, _DPAD), jnp.float32),
        scratch_types=[
            pltpu.VMEM((b_per_w,), jnp.int32),
            pltpu.VMEM((2, b_per_w // 2, D), jnp.float32),
            pltpu.SemaphoreType.DMA,
            pltpu.SemaphoreType.DMA,
        ],
        compiler_params=pltpu.CompilerParams(
            skip_device_barrier=True, use_tc_tiling_on_sc=False
        ),
    )
    def gather_kernel(table_hbm, idx_hbm, out_hbm, idx_v, rows_v, gsem, wsem):
        wid = lax.axis_index("s") * NC + lax.axis_index("c")
        base = wid * b_per_w
        pltpu.sync_copy(idx_hbm.at[pl.ds(base, b_per_w)], idx_v)
        q = b_per_w // 2
        gathers = [
            pltpu.async_copy(
                table_hbm.at[idx_v.at[pl.ds(k * q, q)]], rows_v.at[k], gsem
            )
            for k in range(2)
        ]
        writes = []
        for k in range(2):
            gathers[k].wait()
            writes.append(
                pltpu.async_copy(
                    rows_v.at[k],
                    out_hbm.at[pl.ds(base + k * q, q), pl.ds(0, D)],
                    wsem,
                )
            )
        for w in writes:
            w.wait()

    return gather_kernel


@jax.jit
def kernel(spk_ids, table):
    gather = _make_gather(_NUM_SPEAKERS, _DIM, _BATCH)
    table_pad = jnp.pad(table, ((0, 0), (0, _DPAD - _DIM)))
    table_half = table_pad.reshape(2 * _NUM_SPEAKERS, _DIM)
    out_pad = gather(table_half, spk_ids.astype(jnp.int32) << 1)
    return out_pad[:, :_DIM]
